# baseline (speedup 1.0000x reference)
import functools

import jax
import jax.numpy as jnp
import numpy as np
from jax import lax
from jax.experimental import pallas as pl
from jax.experimental.pallas import tpu as pltpu
from jax.experimental.pallas import tpu_sc as plsc

N = 10000
E = 320000
D = 128

NC = 2
NS = 16
NW = NC * NS
B = 128
NSUB = 4
BSUB = B // NSUB
K0 = 80
K1 = 80
CH = 40
NPH = K0 // CH
E_PAD = NS * (K0 + K1) * B
NPAD = 10112
ROWS_T = NPAD // NS

_PAD_SRC = np.arange(E_PAD - E, dtype=np.int32) % N
_PAD_DST = N + np.arange(E_PAD - E, dtype=np.int32) % (NPAD - N)


def _sc_agg_body(x_hbm, src_hbm, dst_hbm, z_hbm, out_hbm,
                 src_v, dst_v, rows0, rows1, acc_sh, sem0, sem1, ssem0, ssem1):
    c = lax.axis_index("c")
    s = lax.axis_index("s")
    base = jnp.where(c == 0, s * K0, NS * K0 + s * K1)
    nb = jnp.where(c == 0, K0, K1)
    pltpu.sync_copy(z_hbm, acc_sh.at[pl.ds(s * ROWS_T, ROWS_T)])
    plsc.subcore_barrier()

    def fire(j, buf, sem):
        for q in range(NSUB):
            pltpu.async_copy(x_hbm.at[src_v.at[j, pl.ds(q * BSUB, BSUB)]],
                             buf.at[pl.ds(q * BSUB, BSUB)], sem)

    def drain(buf, sem):
        pltpu.make_async_copy(x_hbm.at[src_v.at[0]], buf, sem).wait()

    def drain_scat(buf, sem):
        pltpu.make_async_copy(buf, acc_sh.at[dst_v.at[0]], sem).wait()

    for ph in range(NPH):
        @pl.when(ph * CH < nb)
        def _():
            pltpu.sync_copy(src_hbm.at[pl.ds(base + ph * CH, CH)], src_v)
            pltpu.sync_copy(dst_hbm.at[pl.ds(base + ph * CH, CH)], dst_v)
            fire(0, rows0, sem0)
            fire(1, rows1, sem1)

            def step(i, carry):
                j0 = 2 * i
                j1 = j0 + 1
                drain(rows0, sem0)
                pltpu.async_copy(rows0, acc_sh.at[dst_v.at[j0]], ssem0,
                                 add=True)
                drain(rows1, sem1)
                pltpu.async_copy(rows1, acc_sh.at[dst_v.at[j1]], ssem1,
                                 add=True)
                drain_scat(rows0, ssem0)

                @pl.when(i < CH // 2 - 1)
                def _():
                    fire(j0 + 2, rows0, sem0)

                drain_scat(rows1, ssem1)

                @pl.when(i < CH // 2 - 1)
                def _():
                    fire(j1 + 2, rows1, sem1)

                return carry

            lax.fori_loop(0, CH // 2, step, 0)

    plsc.subcore_barrier()
    pltpu.sync_copy(acc_sh.at[pl.ds(s * ROWS_T, ROWS_T)],
                    out_hbm.at[c, pl.ds(s * ROWS_T, ROWS_T)])


_sc_agg = pl.kernel(
    _sc_agg_body,
    mesh=plsc.VectorSubcoreMesh(core_axis_name="c", subcore_axis_name="s"),
    out_type=jax.ShapeDtypeStruct((NC, NPAD, D), jnp.float32),
    scratch_types=[
        pltpu.VMEM((CH, B), jnp.int32),
        pltpu.VMEM((CH, B), jnp.int32),
        pltpu.VMEM((B, D), jnp.float32),
        pltpu.VMEM((B, D), jnp.float32),
        pltpu.VMEM_SHARED((NPAD, D), jnp.float32),
        pltpu.SemaphoreType.DMA,
        pltpu.SemaphoreType.DMA,
        pltpu.SemaphoreType.DMA,
        pltpu.SemaphoreType.DMA,
    ],
)


def _tc_body(x_b, a0_b, a1_b, bnd_b, rw_b, w1_b, w2_b, bias_b, g_b, be_b, o_b):
    h2 = (a0_b[0] + a1_b[0]) * rw_b[...] + bnd_b[...]
    acc = jnp.dot(x_b[...], w1_b[...], preferred_element_type=jnp.float32)
    acc = acc + jnp.dot(h2, w2_b[...], preferred_element_type=jnp.float32)
    acc = acc + bias_b[...]
    mu = jnp.mean(acc, axis=1, keepdims=True)
    var = jnp.mean(jnp.square(acc - mu), axis=1, keepdims=True)
    y = (acc - mu) * lax.rsqrt(var + 1e-5)
    y = y * g_b[...] + be_b[...]
    o_b[...] = jnp.maximum(y, 0.0)


_R = 1000


def _tc_stage(x, agg, boundary, rw, w1t, w2t, bias, gamma, beta):
    row_spec = pl.BlockSpec((_R, D), lambda i: (i, 0))
    agg0_spec = pl.BlockSpec((1, _R, D), lambda i: (0, i, 0))
    agg1_spec = pl.BlockSpec((1, _R, D), lambda i: (1, i, 0))
    full_spec = pl.BlockSpec((D, D), lambda i: (0, 0))
    vec_spec = pl.BlockSpec((1, D), lambda i: (0, 0))
    return pl.pallas_call(
        _tc_body,
        grid=(N // _R,),
        in_specs=[row_spec, agg0_spec, agg1_spec, row_spec,
                  vec_spec, full_spec, full_spec, vec_spec, vec_spec, vec_spec],
        out_specs=row_spec,
        out_shape=jax.ShapeDtypeStruct((N, D), jnp.float32),
    )(x, agg, agg, boundary, rw, w1t, w2t, bias, gamma, beta)


def kernel(x, boundary, edge_index, relation_weight, W, b, gamma, beta):
    src = edge_index[0]
    dst = edge_index[1]
    src_p = jnp.concatenate([src, jnp.asarray(_PAD_SRC)]).reshape(-1, B)
    dst_p = jnp.concatenate([dst, jnp.asarray(_PAD_DST)]).reshape(-1, B)
    zeros = jnp.zeros((ROWS_T, D), jnp.float32)

    agg = _sc_agg(x, src_p, dst_p, zeros)

    w1t = W[:, :D].T
    w2t = W[:, D:].T
    return _tc_stage(x, agg, boundary,
                     relation_weight.reshape(1, D), w1t, w2t,
                     b.reshape(1, D), gamma.reshape(1, D), beta.reshape(1, D))


# speedup vs baseline: 1.1315x; 1.1315x over previous
import functools

import jax
import jax.numpy as jnp
import numpy as np
from jax import lax
from jax.experimental import pallas as pl
from jax.experimental.pallas import tpu as pltpu
from jax.experimental.pallas import tpu_sc as plsc

N = 10000
E = 320000
D = 128

NC = 2
NS = 16
NW = NC * NS
B = 128
NSUB = 4
BSUB = B // NSUB
K0 = 80
K1 = 80
CH = 40
NPH = K0 // CH
E_PAD = NS * (K0 + K1) * B
NPAD = 10112
ROWS_T = NPAD // NS

_PAD2 = np.stack([np.arange(E_PAD - E, dtype=np.int32) % N,
                  N + np.arange(E_PAD - E, dtype=np.int32) % (NPAD - N)])


def _sc_agg_body(x_hbm, ei_hbm, z_hbm, out_hbm,
                 src_v, dst_v, rows0, rows1, acc_sh, sem0, sem1):
    c = lax.axis_index("c")
    s = lax.axis_index("s")
    base = jnp.where(c == 0, s * K0, NS * K0 + s * K1)
    nb = jnp.where(c == 0, K0, K1)
    pltpu.sync_copy(z_hbm, acc_sh.at[pl.ds(s * ROWS_T, ROWS_T)])
    plsc.subcore_barrier()

    def fire(j, buf, sem):
        for q in range(NSUB):
            pltpu.async_copy(x_hbm.at[src_v.at[j, pl.ds(q * BSUB, BSUB)]],
                             buf.at[pl.ds(q * BSUB, BSUB)], sem)

    def drain(buf, sem):
        pltpu.make_async_copy(x_hbm.at[src_v.at[0]], buf, sem).wait()

    for ph in range(NPH):
        @pl.when(ph * CH < nb)
        def _():
            pltpu.sync_copy(ei_hbm.at[0, pl.ds(base + ph * CH, CH)], src_v)
            pltpu.sync_copy(ei_hbm.at[1, pl.ds(base + ph * CH, CH)], dst_v)
            fire(0, rows0, sem0)

            def step(i, carry):
                j0 = 2 * i
                j1 = j0 + 1
                drain(rows0, sem0)
                fire(j1, rows1, sem1)
                pltpu.sync_copy(rows0, acc_sh.at[dst_v.at[j0]], add=True)
                drain(rows1, sem1)

                @pl.when(i < CH // 2 - 1)
                def _():
                    fire(j0 + 2, rows0, sem0)

                pltpu.sync_copy(rows1, acc_sh.at[dst_v.at[j1]], add=True)
                return carry

            lax.fori_loop(0, CH // 2, step, 0)

    plsc.subcore_barrier()
    pltpu.sync_copy(acc_sh.at[pl.ds(s * ROWS_T, ROWS_T)],
                    out_hbm.at[c, pl.ds(s * ROWS_T, ROWS_T)])


_sc_agg = pl.kernel(
    _sc_agg_body,
    mesh=plsc.VectorSubcoreMesh(core_axis_name="c", subcore_axis_name="s"),
    out_type=jax.ShapeDtypeStruct((NC, NPAD, D), jnp.float32),
    scratch_types=[
        pltpu.VMEM((CH, B), jnp.int32),
        pltpu.VMEM((CH, B), jnp.int32),
        pltpu.VMEM((B, D), jnp.float32),
        pltpu.VMEM((B, D), jnp.float32),
        pltpu.VMEM_SHARED((NPAD, D), jnp.float32),
        pltpu.SemaphoreType.DMA,
        pltpu.SemaphoreType.DMA,
    ],
)


def _tc_body(x_b, a0_b, a1_b, bnd_b, rw_b, w1_b, w2_b, bias_b, g_b, be_b, o_b):
    h2 = (a0_b[0] + a1_b[0]) * rw_b[...] + bnd_b[...]
    acc = jnp.dot(x_b[...], w1_b[...], preferred_element_type=jnp.float32)
    acc = acc + jnp.dot(h2, w2_b[...], preferred_element_type=jnp.float32)
    acc = acc + bias_b[...]
    mu = jnp.mean(acc, axis=1, keepdims=True)
    var = jnp.mean(jnp.square(acc - mu), axis=1, keepdims=True)
    y = (acc - mu) * lax.rsqrt(var + 1e-5)
    y = y * g_b[...] + be_b[...]
    o_b[...] = jnp.maximum(y, 0.0)


_R = 1000


def _tc_stage(x, agg, boundary, rw, w1t, w2t, bias, gamma, beta):
    row_spec = pl.BlockSpec((_R, D), lambda i: (i, 0))
    agg0_spec = pl.BlockSpec((1, _R, D), lambda i: (0, i, 0))
    agg1_spec = pl.BlockSpec((1, _R, D), lambda i: (1, i, 0))
    full_spec = pl.BlockSpec((D, D), lambda i: (0, 0))
    vec_spec = pl.BlockSpec((1, D), lambda i: (0, 0))
    return pl.pallas_call(
        _tc_body,
        grid=(N // _R,),
        in_specs=[row_spec, agg0_spec, agg1_spec, row_spec,
                  vec_spec, full_spec, full_spec, vec_spec, vec_spec, vec_spec],
        out_specs=row_spec,
        out_shape=jax.ShapeDtypeStruct((N, D), jnp.float32),
    )(x, agg, agg, boundary, rw, w1t, w2t, bias, gamma, beta)


def kernel(x, boundary, edge_index, relation_weight, W, b, gamma, beta):
    ei_p = jnp.concatenate([edge_index, jnp.asarray(_PAD2)],
                           axis=1).reshape(2, -1, B)
    zeros = jnp.zeros((ROWS_T, D), jnp.float32)

    agg = _sc_agg(x, ei_p, zeros)

    w1t = W[:, :D].T
    w2t = W[:, D:].T
    return _tc_stage(x, agg, boundary,
                     relation_weight.reshape(1, D), w1t, w2t,
                     b.reshape(1, D), gamma.reshape(1, D), beta.reshape(1, D))


# speedup vs baseline: 1.1501x; 1.0164x over previous
import functools

import jax
import jax.numpy as jnp
import numpy as np
from jax import lax
from jax.experimental import pallas as pl
from jax.experimental.pallas import tpu as pltpu
from jax.experimental.pallas import tpu_sc as plsc

N = 10000
E = 320000
D = 128

NC = 2
NS = 16
NW = NC * NS
B = 128
NSUB = 4
BSUB = B // NSUB
K0 = 80
K1 = 80
CH = 40
NPH = K0 // CH
E_PAD = NS * (K0 + K1) * B
NPAD = 10112
ROWS_T = NPAD // NS

_PAD2 = np.stack([np.arange(E_PAD - E, dtype=np.int32) % N,
                  N + np.arange(E_PAD - E, dtype=np.int32) % (NPAD - N)])


def _sc_agg_body(x_hbm, ei_hbm, z_hbm, out_hbm,
                 src_v, dst_v, rows0, rows1, acc_sh, sem0, sem1):
    c = lax.axis_index("c")
    s = lax.axis_index("s")
    base = jnp.where(c == 0, s * K0, NS * K0 + s * K1)
    nb = jnp.where(c == 0, K0, K1)
    pltpu.sync_copy(z_hbm, acc_sh.at[pl.ds(s * ROWS_T, ROWS_T)])
    plsc.subcore_barrier()

    def fire(j, buf, sem):
        for q in range(NSUB):
            pltpu.async_copy(x_hbm.at[src_v.at[j, pl.ds(q * BSUB, BSUB)]],
                             buf.at[pl.ds(q * BSUB, BSUB)], sem)

    def drain(buf, sem):
        pltpu.make_async_copy(x_hbm.at[src_v.at[0]], buf, sem).wait()

    for ph in range(NPH):
        @pl.when(ph * CH < nb)
        def _():
            pltpu.sync_copy(ei_hbm.at[0, pl.ds(base + ph * CH, CH)], src_v)
            pltpu.sync_copy(ei_hbm.at[1, pl.ds(base + ph * CH, CH)], dst_v)
            fire(0, rows0, sem0)

            def step(i, carry):
                j0 = 2 * i
                j1 = j0 + 1
                drain(rows0, sem0)
                fire(j1, rows1, sem1)
                pltpu.sync_copy(rows0, acc_sh.at[dst_v.at[j0]], add=True)
                drain(rows1, sem1)

                @pl.when(i < CH // 2 - 1)
                def _():
                    fire(j0 + 2, rows0, sem0)

                pltpu.sync_copy(rows1, acc_sh.at[dst_v.at[j1]], add=True)
                return carry

            lax.fori_loop(0, CH // 2, step, 0)

    plsc.subcore_barrier()
    pltpu.sync_copy(acc_sh.at[pl.ds(s * ROWS_T, ROWS_T)],
                    out_hbm.at[c, pl.ds(s * ROWS_T, ROWS_T)])


_sc_agg = pl.kernel(
    _sc_agg_body,
    mesh=plsc.VectorSubcoreMesh(core_axis_name="c", subcore_axis_name="s"),
    out_type=jax.ShapeDtypeStruct((NC, NPAD, D), jnp.float32),
    scratch_types=[
        pltpu.VMEM((CH, B), jnp.int32),
        pltpu.VMEM((CH, B), jnp.int32),
        pltpu.VMEM((B, D), jnp.float32),
        pltpu.VMEM((B, D), jnp.float32),
        pltpu.VMEM_SHARED((NPAD, D), jnp.float32),
        pltpu.SemaphoreType.DMA,
        pltpu.SemaphoreType.DMA,
    ],
)


def _tc_body(x_b, a0_b, a1_b, bnd_b, rw_b, w1_b, w2_b, bias_b, g_b, be_b, o_b):
    h2 = (a0_b[0] + a1_b[0]) * rw_b[...] + bnd_b[...]
    acc = jnp.dot(x_b[...], w1_b[...], preferred_element_type=jnp.float32)
    acc = acc + jnp.dot(h2, w2_b[...], preferred_element_type=jnp.float32)
    acc = acc + bias_b[...]
    mu = jnp.mean(acc, axis=1, keepdims=True)
    var = jnp.mean(jnp.square(acc - mu), axis=1, keepdims=True)
    y = (acc - mu) * lax.rsqrt(var + 1e-5)
    y = y * g_b[...] + be_b[...]
    o_b[...] = jnp.maximum(y, 0.0)


_R = 2000


def _tc_stage(x, agg, boundary, rw, w1t, w2t, bias, gamma, beta):
    row_spec = pl.BlockSpec((_R, D), lambda i: (i, 0))
    agg0_spec = pl.BlockSpec((1, _R, D), lambda i: (0, i, 0))
    agg1_spec = pl.BlockSpec((1, _R, D), lambda i: (1, i, 0))
    full_spec = pl.BlockSpec((D, D), lambda i: (0, 0))
    vec_spec = pl.BlockSpec((1, D), lambda i: (0, 0))
    return pl.pallas_call(
        _tc_body,
        grid=(N // _R,),
        in_specs=[row_spec, agg0_spec, agg1_spec, row_spec,
                  vec_spec, full_spec, full_spec, vec_spec, vec_spec, vec_spec],
        out_specs=row_spec,
        out_shape=jax.ShapeDtypeStruct((N, D), jnp.float32),
    )(x, agg, agg, boundary, rw, w1t, w2t, bias, gamma, beta)


def kernel(x, boundary, edge_index, relation_weight, W, b, gamma, beta):
    ei_p = jnp.concatenate([edge_index, jnp.asarray(_PAD2)],
                           axis=1).reshape(2, -1, B)
    zeros = jnp.zeros((ROWS_T, D), jnp.float32)

    agg = _sc_agg(x, ei_p, zeros)

    w1t = W[:, :D].T
    w2t = W[:, D:].T
    return _tc_stage(x, agg, boundary,
                     relation_weight.reshape(1, D), w1t, w2t,
                     b.reshape(1, D), gamma.reshape(1, D), beta.reshape(1, D))


# speedup vs baseline: 1.1545x; 1.0039x over previous
import functools

import jax
import jax.numpy as jnp
import numpy as np
from jax import lax
from jax.experimental import pallas as pl
from jax.experimental.pallas import tpu as pltpu
from jax.experimental.pallas import tpu_sc as plsc

N = 10000
E = 320000
D = 128

NC = 2
NS = 16
NW = NC * NS
B = 128
NSUB = 1
BSUB = B // NSUB
K0 = 80
K1 = 80
CH = 40
NPH = K0 // CH
E_PAD = NS * (K0 + K1) * B
NPAD = 10112
ROWS_T = NPAD // NS

_PAD2 = np.stack([np.arange(E_PAD - E, dtype=np.int32) % N,
                  N + np.arange(E_PAD - E, dtype=np.int32) % (NPAD - N)])


def _sc_agg_body(x_hbm, ei_hbm, z_hbm, out_hbm,
                 src_v, dst_v, rows0, rows1, acc_sh, sem0, sem1):
    c = lax.axis_index("c")
    s = lax.axis_index("s")
    base = jnp.where(c == 0, s * K0, NS * K0 + s * K1)
    nb = jnp.where(c == 0, K0, K1)
    pltpu.sync_copy(z_hbm, acc_sh.at[pl.ds(s * ROWS_T, ROWS_T)])
    plsc.subcore_barrier()

    def fire(j, buf, sem):
        for q in range(NSUB):
            pltpu.async_copy(x_hbm.at[src_v.at[j, pl.ds(q * BSUB, BSUB)]],
                             buf.at[pl.ds(q * BSUB, BSUB)], sem)

    def drain(buf, sem):
        pltpu.make_async_copy(x_hbm.at[src_v.at[0]], buf, sem).wait()

    for ph in range(NPH):
        @pl.when(ph * CH < nb)
        def _():
            pltpu.sync_copy(ei_hbm.at[0, pl.ds(base + ph * CH, CH)], src_v)
            pltpu.sync_copy(ei_hbm.at[1, pl.ds(base + ph * CH, CH)], dst_v)
            fire(0, rows0, sem0)

            def step(i, carry):
                j0 = 2 * i
                j1 = j0 + 1
                drain(rows0, sem0)
                fire(j1, rows1, sem1)
                pltpu.sync_copy(rows0, acc_sh.at[dst_v.at[j0]], add=True)
                drain(rows1, sem1)

                @pl.when(i < CH // 2 - 1)
                def _():
                    fire(j0 + 2, rows0, sem0)

                pltpu.sync_copy(rows1, acc_sh.at[dst_v.at[j1]], add=True)
                return carry

            lax.fori_loop(0, CH // 2, step, 0)

    plsc.subcore_barrier()
    pltpu.sync_copy(acc_sh.at[pl.ds(s * ROWS_T, ROWS_T)],
                    out_hbm.at[c, pl.ds(s * ROWS_T, ROWS_T)])


_sc_agg = pl.kernel(
    _sc_agg_body,
    mesh=plsc.VectorSubcoreMesh(core_axis_name="c", subcore_axis_name="s"),
    out_type=jax.ShapeDtypeStruct((NC, NPAD, D), jnp.float32),
    scratch_types=[
        pltpu.VMEM((CH, B), jnp.int32),
        pltpu.VMEM((CH, B), jnp.int32),
        pltpu.VMEM((B, D), jnp.float32),
        pltpu.VMEM((B, D), jnp.float32),
        pltpu.VMEM_SHARED((NPAD, D), jnp.float32),
        pltpu.SemaphoreType.DMA,
        pltpu.SemaphoreType.DMA,
    ],
)


def _tc_body(x_b, a0_b, a1_b, bnd_b, rw_b, w1_b, w2_b, bias_b, g_b, be_b, o_b):
    h2 = (a0_b[0] + a1_b[0]) * rw_b[...] + bnd_b[...]
    acc = jnp.dot(x_b[...], w1_b[...], preferred_element_type=jnp.float32)
    acc = acc + jnp.dot(h2, w2_b[...], preferred_element_type=jnp.float32)
    acc = acc + bias_b[...]
    mu = jnp.mean(acc, axis=1, keepdims=True)
    var = jnp.mean(jnp.square(acc - mu), axis=1, keepdims=True)
    y = (acc - mu) * lax.rsqrt(var + 1e-5)
    y = y * g_b[...] + be_b[...]
    o_b[...] = jnp.maximum(y, 0.0)


_R = 2000


def _tc_stage(x, agg, boundary, rw, w1t, w2t, bias, gamma, beta):
    row_spec = pl.BlockSpec((_R, D), lambda i: (i, 0))
    agg0_spec = pl.BlockSpec((1, _R, D), lambda i: (0, i, 0))
    agg1_spec = pl.BlockSpec((1, _R, D), lambda i: (1, i, 0))
    full_spec = pl.BlockSpec((D, D), lambda i: (0, 0))
    vec_spec = pl.BlockSpec((1, D), lambda i: (0, 0))
    return pl.pallas_call(
        _tc_body,
        grid=(N // _R,),
        in_specs=[row_spec, agg0_spec, agg1_spec, row_spec,
                  vec_spec, full_spec, full_spec, vec_spec, vec_spec, vec_spec],
        out_specs=row_spec,
        out_shape=jax.ShapeDtypeStruct((N, D), jnp.float32),
    )(x, agg, agg, boundary, rw, w1t, w2t, bias, gamma, beta)


def kernel(x, boundary, edge_index, relation_weight, W, b, gamma, beta):
    ei_p = jnp.concatenate([edge_index, jnp.asarray(_PAD2)],
                           axis=1).reshape(2, -1, B)
    zeros = jnp.zeros((ROWS_T, D), jnp.float32)

    agg = _sc_agg(x, ei_p, zeros)

    w1t = W[:, :D].T
    w2t = W[:, D:].T
    return _tc_stage(x, agg, boundary,
                     relation_weight.reshape(1, D), w1t, w2t,
                     b.reshape(1, D), gamma.reshape(1, D), beta.reshape(1, D))
